# Initial kernel scaffold; baseline (speedup 1.0000x reference)
import jax
import jax.numpy as jnp
from jax.experimental import pallas as pl


def kernel(x, target, target_mask):
    raise NotImplementedError("write your pallas kernel here")


# speedup vs baseline: 2.3210x; 2.3210x over previous
import functools
import math

import jax
import jax.numpy as jnp
from jax import lax
from jax.experimental import pallas as pl
from jax.experimental.pallas import tpu as pltpu
from jax.experimental.pallas import tpu_sc as plsc

N = 4096
V = 32000
SMOOTHING = 0.1
CONFIDENCE = 1.0 - SMOOTHING
EPS = SMOOTHING / (V - 1)
ROW_CONST = CONFIDENCE * math.log(CONFIDENCE) + SMOOTHING * math.log(EPS)

_BR = 512
_BV = 3200


def _tc_body(m_ref, x_ref, o_ref):
    i = pl.program_id(0)
    j = pl.program_id(1)
    part = jnp.sum(x_ref[...] * m_ref[...][:, None], keepdims=True)

    @pl.when((i == 0) & (j == 0))
    def _init():
        o_ref[...] = jnp.zeros_like(o_ref)

    o_ref[...] += part


_tc_masked_sum = pl.pallas_call(
    _tc_body,
    grid=(N // _BR, V // _BV),
    in_specs=[
        pl.BlockSpec((_BR,), lambda i, j: (i,)),
        pl.BlockSpec((_BR, _BV), lambda i, j: (i, j)),
    ],
    out_specs=pl.BlockSpec((1, 1), lambda i, j: (0, 0)),
    out_shape=jax.ShapeDtypeStruct((1, 1), jnp.float32),
)

_NC = 2
_NS = 16
_NW = _NC * _NS
_RPW = N // _NW


def _sc_make():
    mesh = plsc.VectorSubcoreMesh(core_axis_name="c", subcore_axis_name="s")

    @functools.partial(
        pl.kernel,
        mesh=mesh,
        out_type=[
            jax.ShapeDtypeStruct((_NW, 16), jnp.float32),
            jax.ShapeDtypeStruct((_NW, 16), jnp.float32),
        ],
        scratch_types=[
            pltpu.VMEM((_RPW,), jnp.int32),
            pltpu.VMEM((_RPW,), jnp.int32),
            pltpu.VMEM((_RPW,), jnp.int32),
            pltpu.VMEM((_RPW,), jnp.float32),
            pltpu.VMEM((16,), jnp.float32),
            pltpu.VMEM((16,), jnp.float32),
            pltpu.SemaphoreType.DMA,
        ],
    )
    def sc_gather(x_hbm, tgt_hbm, msk_hbm, g_out, m_out,
                  t_v, m_v, idx_v, g_v, ag_v, am_v, sem):
        wid = lax.axis_index("s") * _NC + lax.axis_index("c")
        base = wid * _RPW
        pltpu.sync_copy(tgt_hbm.at[pl.ds(base, _RPW)], t_v)
        pltpu.sync_copy(msk_hbm.at[pl.ds(base, _RPW)], m_v)
        for i in range(_RPW // 16):
            rows = lax.iota(jnp.int32, 16) + (base + i * 16)
            idx_v[pl.ds(i * 16, 16)] = rows * V + t_v[pl.ds(i * 16, 16)]
        pltpu.async_copy(x_hbm.at[idx_v], g_v, sem).wait()
        ag = jnp.zeros((16,), jnp.float32)
        am = jnp.zeros((16,), jnp.float32)
        for i in range(_RPW // 16):
            mf = m_v[pl.ds(i * 16, 16)].astype(jnp.float32)
            ag = ag + g_v[pl.ds(i * 16, 16)] * mf
            am = am + mf
        ag_v[...] = ag
        am_v[...] = am
        pltpu.sync_copy(ag_v, g_out.at[wid])
        pltpu.sync_copy(am_v, m_out.at[wid])

    return sc_gather


_sc_gather = _sc_make()


def kernel(x, target, target_mask):
    tgt = target.astype(jnp.int32)
    msk = target_mask.astype(jnp.int32)
    g_parts, m_parts = _sc_gather(x.reshape(-1), tgt, msk)
    s_masked = _tc_masked_sum(target_mask.astype(jnp.float32), x)[0, 0]
    g = jnp.sum(g_parts)
    m = jnp.sum(m_parts)
    return m * ROW_CONST - EPS * s_masked - (CONFIDENCE - EPS) * g


# speedup vs baseline: 2.4594x; 1.0597x over previous
import functools
import math

import jax
import jax.numpy as jnp
from jax import lax
from jax.experimental import pallas as pl
from jax.experimental.pallas import tpu as pltpu
from jax.experimental.pallas import tpu_sc as plsc

N = 4096
V = 32000
SMOOTHING = 0.1
CONFIDENCE = 1.0 - SMOOTHING
EPS = SMOOTHING / (V - 1)
ROW_CONST = CONFIDENCE * math.log(CONFIDENCE) + SMOOTHING * math.log(EPS)

_BR = 128


def _tc_body(m_ref, x_ref, o_ref):
    i = pl.program_id(0)
    part = jnp.sum(x_ref[...] * m_ref[...][:, None], keepdims=True)

    @pl.when(i == 0)
    def _init():
        o_ref[...] = jnp.zeros_like(o_ref)

    o_ref[...] += part


_tc_masked_sum = pl.pallas_call(
    _tc_body,
    grid=(N // _BR,),
    in_specs=[
        pl.BlockSpec((_BR,), lambda i: (i,)),
        pl.BlockSpec((_BR, V), lambda i: (i, 0)),
    ],
    out_specs=pl.BlockSpec((1, 1), lambda i: (0, 0)),
    out_shape=jax.ShapeDtypeStruct((1, 1), jnp.float32),
)

_NC = 2
_NS = 16
_NW = _NC * _NS
_RPW = N // _NW


def _sc_make():
    mesh = plsc.VectorSubcoreMesh(core_axis_name="c", subcore_axis_name="s")

    @functools.partial(
        pl.kernel,
        mesh=mesh,
        out_type=[
            jax.ShapeDtypeStruct((_NW, 16), jnp.float32),
            jax.ShapeDtypeStruct((_NW, 16), jnp.float32),
        ],
        scratch_types=[
            pltpu.VMEM((_RPW,), jnp.int32),
            pltpu.VMEM((_RPW,), jnp.int32),
            pltpu.VMEM((_RPW,), jnp.int32),
            pltpu.VMEM((_RPW,), jnp.float32),
            pltpu.VMEM((16,), jnp.float32),
            pltpu.VMEM((16,), jnp.float32),
            pltpu.SemaphoreType.DMA,
        ],
    )
    def sc_gather(x_hbm, tgt_hbm, msk_hbm, g_out, m_out,
                  t_v, m_v, idx_v, g_v, ag_v, am_v, sem):
        wid = lax.axis_index("s") * _NC + lax.axis_index("c")
        base = wid * _RPW
        pltpu.sync_copy(tgt_hbm.at[pl.ds(base, _RPW)], t_v)
        pltpu.sync_copy(msk_hbm.at[pl.ds(base, _RPW)], m_v)
        for i in range(_RPW // 16):
            rows = lax.iota(jnp.int32, 16) + (base + i * 16)
            idx_v[pl.ds(i * 16, 16)] = rows * V + t_v[pl.ds(i * 16, 16)]
        pltpu.async_copy(x_hbm.at[idx_v], g_v, sem).wait()
        ag = jnp.zeros((16,), jnp.float32)
        am = jnp.zeros((16,), jnp.float32)
        for i in range(_RPW // 16):
            mf = m_v[pl.ds(i * 16, 16)].astype(jnp.float32)
            ag = ag + g_v[pl.ds(i * 16, 16)] * mf
            am = am + mf
        ag_v[...] = ag
        am_v[...] = am
        pltpu.sync_copy(ag_v, g_out.at[wid])
        pltpu.sync_copy(am_v, m_out.at[wid])

    return sc_gather


_sc_gather = _sc_make()


def kernel(x, target, target_mask):
    tgt = target.astype(jnp.int32)
    msk = target_mask.astype(jnp.int32)
    g_parts, m_parts = _sc_gather(x.reshape(-1), tgt, msk)
    s_masked = _tc_masked_sum(target_mask.astype(jnp.float32), x)[0, 0]
    g = jnp.sum(g_parts)
    m = jnp.sum(m_parts)
    return m * ROW_CONST - EPS * s_masked - (CONFIDENCE - EPS) * g


# speedup vs baseline: 2.4597x; 1.0001x over previous
import functools
import math

import jax
import jax.numpy as jnp
from jax import lax
from jax.experimental import pallas as pl
from jax.experimental.pallas import tpu as pltpu
from jax.experimental.pallas import tpu_sc as plsc

N = 4096
V = 32000
SMOOTHING = 0.1
CONFIDENCE = 1.0 - SMOOTHING
EPS = SMOOTHING / (V - 1)
ROW_CONST = CONFIDENCE * math.log(CONFIDENCE) + SMOOTHING * math.log(EPS)

_TC_ROWS = 3072

_BR = 128


def _tc_body(m_ref, x_ref, o_ref):
    i = pl.program_id(0)
    part = jnp.sum(x_ref[...] * m_ref[...][:, None], keepdims=True)

    @pl.when(i == 0)
    def _init():
        o_ref[...] = jnp.zeros_like(o_ref)

    o_ref[...] += part


_tc_masked_sum = pl.pallas_call(
    _tc_body,
    grid=(_TC_ROWS // _BR,),
    in_specs=[
        pl.BlockSpec((_BR,), lambda i: (i,)),
        pl.BlockSpec((_BR, V), lambda i: (i, 0)),
    ],
    out_specs=pl.BlockSpec((1, 1), lambda i: (0, 0)),
    out_shape=jax.ShapeDtypeStruct((1, 1), jnp.float32),
)

_NC = 2
_NS = 16
_NW = _NC * _NS
_RPW = N // _NW
_SC_ROWS = N - _TC_ROWS
_DPW = _SC_ROWS // _NW
_UNROLL = 20


def _row_acc(buf, acc):
    def step(k, a):
        base = k * (16 * _UNROLL)
        vals = [buf[pl.ds(base + u * 16, 16)] for u in range(_UNROLL)]
        while len(vals) > 1:
            vals = [vals[i] + vals[i + 1] for i in range(0, len(vals) - 1, 2)] + (
                [vals[-1]] if len(vals) % 2 else [])
        return a + vals[0]

    return lax.fori_loop(0, V // (16 * _UNROLL), step, acc)


def _sc_make():
    mesh = plsc.VectorSubcoreMesh(core_axis_name="c", subcore_axis_name="s")

    @functools.partial(
        pl.kernel,
        mesh=mesh,
        out_type=[
            jax.ShapeDtypeStruct((_NW, 16), jnp.float32),
            jax.ShapeDtypeStruct((_NW, 16), jnp.float32),
            jax.ShapeDtypeStruct((_NW, 16), jnp.float32),
        ],
        scratch_types=[
            pltpu.VMEM((_RPW,), jnp.int32),
            pltpu.VMEM((_RPW,), jnp.int32),
            pltpu.VMEM((_RPW,), jnp.int32),
            pltpu.VMEM((_RPW,), jnp.float32),
            pltpu.VMEM((_DPW * 16,), jnp.float32),
            pltpu.VMEM((V,), jnp.float32),
            pltpu.VMEM((V,), jnp.float32),
            pltpu.VMEM((16,), jnp.float32),
            pltpu.SemaphoreType.DMA,
            pltpu.SemaphoreType.DMA,
            pltpu.SemaphoreType.DMA,
        ],
    )
    def sc_gather(x_hbm, x2_hbm, tgt_hbm, msk_hbm, mrep_hbm, g_out, m_out, s_out,
                  t_v, m_v, idx_v, g_v, dmf_v, buf0, buf1, st_v,
                  gsem, sem0, sem1):
        wid = lax.axis_index("s") * _NC + lax.axis_index("c")

        base = wid * _RPW
        pltpu.sync_copy(tgt_hbm.at[pl.ds(base, _RPW)], t_v)
        pltpu.sync_copy(msk_hbm.at[pl.ds(base, _RPW)], m_v)
        for i in range(_RPW // 16):
            rows = lax.iota(jnp.int32, 16) + (base + i * 16)
            idx_v[pl.ds(i * 16, 16)] = rows * V + t_v[pl.ds(i * 16, 16)]
        pltpu.async_copy(x_hbm.at[idx_v], g_v, gsem).wait()
        ag = jnp.zeros((16,), jnp.float32)
        am = jnp.zeros((16,), jnp.float32)
        for i in range(_RPW // 16):
            mf = m_v[pl.ds(i * 16, 16)].astype(jnp.float32)
            ag = ag + g_v[pl.ds(i * 16, 16)] * mf
            am = am + mf
        st_v[...] = ag
        pltpu.sync_copy(st_v, g_out.at[wid])
        st_v[...] = am
        pltpu.sync_copy(st_v, m_out.at[wid])

        dbase = _TC_ROWS + wid * _DPW
        pltpu.sync_copy(mrep_hbm.at[pl.ds(wid * _DPW * 16, _DPW * 16)], dmf_v)

        pltpu.async_copy(x2_hbm.at[dbase], buf0, sem0)

        def pair(j2, acc):
            ra = dbase + 2 * j2
            pltpu.make_async_copy(x2_hbm.at[ra], buf0, sem0).wait()
            pltpu.async_copy(x2_hbm.at[ra + 1], buf1, sem1)
            r = _row_acc(buf0, jnp.zeros((16,), jnp.float32))
            acc = acc + r * dmf_v[pl.ds(2 * j2 * 16, 16)]

            pltpu.make_async_copy(x2_hbm.at[ra + 1], buf1, sem1).wait()

            @pl.when(j2 + 1 < _DPW // 2)
            def _prefetch():
                pltpu.async_copy(x2_hbm.at[ra + 2], buf0, sem0)

            r = _row_acc(buf1, jnp.zeros((16,), jnp.float32))
            return acc + r * dmf_v[pl.ds((2 * j2 + 1) * 16, 16)]

        asum = lax.fori_loop(0, _DPW // 2, pair, jnp.zeros((16,), jnp.float32))
        st_v[...] = asum
        pltpu.sync_copy(st_v, s_out.at[wid])

    return sc_gather


_sc_gather = _sc_make()


def kernel(x, target, target_mask):
    tgt = target.astype(jnp.int32)
    msk = target_mask.astype(jnp.int32)
    mrep = jnp.repeat(target_mask[_TC_ROWS:].astype(jnp.float32), 16)
    g_parts, m_parts, s_parts = _sc_gather(x.reshape(-1), x, tgt, msk, mrep)
    s_tc = _tc_masked_sum(target_mask.astype(jnp.float32), x)[0, 0]
    s_masked = s_tc + jnp.sum(s_parts)
    g = jnp.sum(g_parts)
    m = jnp.sum(m_parts)
    return m * ROW_CONST - EPS * s_masked - (CONFIDENCE - EPS) * g
